# baseline (speedup 1.0000x reference)
import jax
import jax.numpy as jnp
from jax import lax
from jax.experimental import pallas as pl
from jax.experimental.pallas import tpu as pltpu
from jax.experimental.pallas import tpu_sc as plsc

N = 10000
E = 320000
F_IN = 128
LATENT = 128
CLASSES = 16

NC = 2
NS = 16
NW = NC * NS
K = 64
NCHUNK = 160
EPW = K * NCHUNK
EPAD = NW * EPW

ROWS_T = 624
ROWS_LAST = N - 15 * ROWS_T


def _broadcast_lane(vec, e):
    idx = jnp.full((16, 1), e, dtype=jnp.int32)
    dn = lax.GatherDimensionNumbers(
        offset_dims=(), collapsed_slice_dims=(0,), start_index_map=(0,))
    return lax.gather(vec, idx, dn, slice_sizes=(1,),
                      mode=lax.GatherScatterMode.PROMISE_IN_BOUNDS)


def _make_sc_aggregate(feat, npass):
    nfv = feat // 16

    def body(h_hbm, src_hbm, dst_hbm, w_hbm, zeros_hbm, out_hbm,
             acc, table, srcs, dsts, ws, rows,
             sg0, sg1, sg2, ss0, ss1, ss2):
        c = lax.axis_index("c")
        s = lax.axis_index("s")
        wid = s * NC + c
        sems_g = (sg0, sg1, sg2)
        sems_s = (ss0, ss1, ss2)
        r0 = s * ROWS_T
        rext = pl.ds(16 * ROWS_T, ROWS_LAST - ROWS_T)

        pltpu.sync_copy(src_hbm.at[wid], srcs)
        pltpu.sync_copy(dst_hbm.at[wid], dsts)
        pltpu.sync_copy(w_hbm.at[wid], ws)

        def gather_start(j, b):
            pltpu.async_copy(table.at[srcs.at[j]], rows.at[b], sems_g[b])

        def gather_wait(j, b):
            pltpu.make_async_copy(table.at[srcs.at[j]], rows.at[b],
                                  sems_g[b]).wait()

        def scatter_start(j, b):
            pltpu.async_copy(rows.at[b], acc.at[dsts.at[j]], sems_s[b],
                             add=True)

        def scatter_wait(j, b):
            pltpu.make_async_copy(rows.at[b], acc.at[dsts.at[j]],
                                  sems_s[b]).wait()

        def compute(j, b):
            for g in range(K // 16):
                wv = ws[j, pl.ds(g * 16, 16)]
                for e in range(16):
                    wb = _broadcast_lane(wv, e)
                    ge = g * 16 + e
                    for f in range(nfv):
                        sl = pl.ds(f * 16, 16)
                        rows[b, ge, sl] = rows[b, ge, sl] * wb

        for p in range(npass):
            pltpu.sync_copy(h_hbm.at[p, pl.ds(r0, ROWS_T)],
                            table.at[pl.ds(r0, ROWS_T)])
            pltpu.sync_copy(zeros_hbm.at[pl.ds(r0, ROWS_T)],
                            acc.at[pl.ds(r0, ROWS_T)])

            @pl.when(s == NS - 1)
            def _():
                pltpu.sync_copy(h_hbm.at[p, rext], table.at[rext])
                pltpu.sync_copy(zeros_hbm.at[rext], acc.at[rext])

            plsc.subcore_barrier()

            gather_start(0, 0)
            gather_start(1, 1)

            def triple(jj, _):
                for b in range(3):
                    j = 3 * jj + b
                    gather_wait(j, b)

                    @pl.when(j >= 1)
                    def _():
                        scatter_wait(j - 1, (b + 2) % 3)

                    @pl.when(j + 2 < NCHUNK)
                    def _():
                        gather_start(j + 2, (b + 2) % 3)

                    compute(j, b)
                    scatter_start(j, b)
                return ()

            lax.fori_loop(0, NCHUNK // 3, triple, ())
            for j in range(3 * (NCHUNK // 3), NCHUNK):
                b = j % 3
                gather_wait(j, b)
                scatter_wait(j - 1, (b + 2) % 3)
                compute(j, b)
                scatter_start(j, b)
            scatter_wait(NCHUNK - 1, (NCHUNK - 1) % 3)

            plsc.subcore_barrier()

            pltpu.sync_copy(acc.at[pl.ds(r0, ROWS_T)],
                            out_hbm.at[p, c, pl.ds(r0, ROWS_T)])

            @pl.when(s == NS - 1)
            def _():
                pltpu.sync_copy(acc.at[rext], out_hbm.at[p, c, rext])

    mesh = plsc.VectorSubcoreMesh(core_axis_name="c", subcore_axis_name="s")
    return pl.kernel(
        body,
        out_type=jax.ShapeDtypeStruct((npass, NC, N, feat), jnp.float32),
        mesh=mesh,
        compiler_params=pltpu.CompilerParams(use_tc_tiling_on_sc=False),
        scratch_types=[
            pltpu.VMEM_SHARED((N, feat), jnp.float32),
            pltpu.VMEM_SHARED((N, feat), jnp.float32),
            pltpu.VMEM((NCHUNK, K), jnp.int32),
            pltpu.VMEM((NCHUNK, K), jnp.int32),
            pltpu.VMEM((NCHUNK, K), jnp.float32),
            pltpu.VMEM((3, K, feat), jnp.float32),
            pltpu.SemaphoreType.DMA,
            pltpu.SemaphoreType.DMA,
            pltpu.SemaphoreType.DMA,
            pltpu.SemaphoreType.DMA,
            pltpu.SemaphoreType.DMA,
            pltpu.SemaphoreType.DMA,
        ],
    )


_sc_agg_64x2 = _make_sc_aggregate(LATENT // 2, 2)
_sc_agg_16 = _make_sc_aggregate(CLASSES, 1)

HALF = LATENT // 2


def _mm_body(x_ref, w_ref, o_ref):
    o_ref[...] = jnp.dot(x_ref[...], w_ref[...],
                         preferred_element_type=jnp.float32)


def _matmul_split(x, w):
    n, k = x.shape
    blk = 400

    def mm_body(x_ref, w_ref, o_ref):
        o_ref[0, ...] = jnp.dot(x_ref[...], w_ref[0],
                                preferred_element_type=jnp.float32)

    return pl.pallas_call(
        mm_body,
        grid=(n // blk, 2),
        in_specs=[pl.BlockSpec((blk, k), lambda i, j: (i, 0)),
                  pl.BlockSpec((1, k, HALF), lambda i, j: (j, 0, 0))],
        out_specs=pl.BlockSpec((1, blk, HALF), lambda i, j: (j, i, 0)),
        out_shape=jax.ShapeDtypeStruct((2, n, HALF), jnp.float32),
    )(x, w.reshape(k, 2, HALF).transpose(1, 0, 2))


def _combine1_body(pa0_ref, pa1_ref, pb0_ref, pb1_ref, b_ref,
                   w2a_ref, w2b_ref, out1_ref, ixz_ref, h2_ref):
    oa = pa0_ref[0, 0] + pa1_ref[0, 0] + b_ref[0, 0]
    ob = pb0_ref[0, 0] + pb1_ref[0, 0] + b_ref[0, 1]
    out1_ref[...] = jnp.concatenate([oa, ob], axis=1)
    std = jax.nn.softplus(ob) + 1e-10
    ixz_ref[...] = -jnp.log(std) + (std * std + oa * oa) / 2.0 - 0.5
    h2_ref[...] = (jnp.dot(oa, w2a_ref[0], preferred_element_type=jnp.float32)
                   + jnp.dot(ob, w2b_ref[0], preferred_element_type=jnp.float32))


def _combine1(p, b1, W2):
    blk = 400
    return pl.pallas_call(
        _combine1_body,
        grid=(N // blk,),
        in_specs=[pl.BlockSpec((1, 1, blk, HALF), lambda i: (0, 0, i, 0)),
                  pl.BlockSpec((1, 1, blk, HALF), lambda i: (0, 1, i, 0)),
                  pl.BlockSpec((1, 1, blk, HALF), lambda i: (1, 0, i, 0)),
                  pl.BlockSpec((1, 1, blk, HALF), lambda i: (1, 1, i, 0)),
                  pl.BlockSpec((1, 2, HALF), lambda i: (0, 0, 0)),
                  pl.BlockSpec((1, HALF, CLASSES), lambda i: (0, 0, 0)),
                  pl.BlockSpec((1, HALF, CLASSES), lambda i: (1, 0, 0))],
        out_specs=[pl.BlockSpec((blk, LATENT), lambda i: (i, 0)),
                   pl.BlockSpec((blk, HALF), lambda i: (i, 0)),
                   pl.BlockSpec((blk, CLASSES), lambda i: (i, 0))],
        out_shape=[jax.ShapeDtypeStruct((N, LATENT), jnp.float32),
                   jax.ShapeDtypeStruct((N, HALF), jnp.float32),
                   jax.ShapeDtypeStruct((N, CLASSES), jnp.float32)],
    )(p, p, p, p,
      b1.reshape(1, 2, HALF),
      W2.reshape(2, HALF, CLASSES), W2.reshape(2, HALF, CLASSES))


def _combine2_body(p0_ref, p1_ref, b_ref, out2_ref, ixz_ref):
    o = p0_ref[0, 0] + p1_ref[0, 0] + b_ref[...]
    out2_ref[...] = o
    mean = o[:, :CLASSES // 2]
    std = jax.nn.softplus(o[:, CLASSES // 2:]) + 1e-10
    ixz_ref[...] = -jnp.log(std) + (std * std + mean * mean) / 2.0 - 0.5


def _combine2(p, b2):
    blk = 1000
    return pl.pallas_call(
        _combine2_body,
        grid=(N // blk,),
        in_specs=[pl.BlockSpec((1, 1, blk, CLASSES), lambda i: (0, 0, i, 0)),
                  pl.BlockSpec((1, 1, blk, CLASSES), lambda i: (0, 1, i, 0)),
                  pl.BlockSpec((1, CLASSES), lambda i: (0, 0))],
        out_specs=[pl.BlockSpec((blk, CLASSES), lambda i: (i, 0)),
                   pl.BlockSpec((blk, CLASSES // 2), lambda i: (i, 0))],
        out_shape=[jax.ShapeDtypeStruct((N, CLASSES), jnp.float32),
                   jax.ShapeDtypeStruct((N, CLASSES // 2), jnp.float32)],
    )(p, p, b2.reshape(1, CLASSES))


def kernel(x, edge_index, edge_attr, W1, b1, W2, b2):
    pad = EPAD - E
    src = jnp.pad(edge_index[0].astype(jnp.int32), (0, pad)).reshape(NW, NCHUNK, K)
    dst = jnp.pad(edge_index[1].astype(jnp.int32), (0, pad)).reshape(NW, NCHUNK, K)
    w = jnp.pad(edge_attr.astype(jnp.float32), (0, pad)).reshape(NW, NCHUNK, K)
    zeros64 = jnp.zeros((N, HALF), jnp.float32)
    zeros16 = jnp.zeros((N, CLASSES), jnp.float32)

    h1s = _matmul_split(x, W1)
    p1 = _sc_agg_64x2(h1s, src, dst, w, zeros64)
    out1, ixz1, h2 = _combine1(p1, b1, W2)
    p2 = _sc_agg_16(h2[None], src, dst, w, zeros16)
    out2, ixz2 = _combine2(p2, b2)

    skl1 = jnp.zeros_like(ixz1)
    skl2 = jnp.zeros_like(ixz2)
    return (out2, out1, ixz1, skl1, ixz2, skl2)


# speedup vs baseline: 1.0039x; 1.0039x over previous
import jax
import jax.numpy as jnp
from jax import lax
from jax.experimental import pallas as pl
from jax.experimental.pallas import tpu as pltpu
from jax.experimental.pallas import tpu_sc as plsc

N = 10000
E = 320000
F_IN = 128
LATENT = 128
CLASSES = 16

NC = 2
NS = 16
NW = NC * NS
K = 64
NCHUNK = 160
EPW = K * NCHUNK
EPAD = NW * EPW

ROWS_T = 624
ROWS_LAST = N - 15 * ROWS_T


def _broadcast_lane(vec, e):
    idx = jnp.full((16, 1), e, dtype=jnp.int32)
    dn = lax.GatherDimensionNumbers(
        offset_dims=(), collapsed_slice_dims=(0,), start_index_map=(0,))
    return lax.gather(vec, idx, dn, slice_sizes=(1,),
                      mode=lax.GatherScatterMode.PROMISE_IN_BOUNDS)


def _make_sc_aggregate(feat, npass):
    nfv = feat // 16

    def body(h_hbm, src_hbm, dst_hbm, w_hbm, zeros_hbm, out_hbm,
             acc, table, srcs, dsts, ws, rows, sg0, sg1, ss0, ss1):
        c = lax.axis_index("c")
        s = lax.axis_index("s")
        wid = s * NC + c
        sems_g = (sg0, sg1)
        sems_s = (ss0, ss1)
        r0 = s * ROWS_T
        rext = pl.ds(16 * ROWS_T, ROWS_LAST - ROWS_T)

        pltpu.sync_copy(src_hbm.at[wid], srcs)
        pltpu.sync_copy(dst_hbm.at[wid], dsts)
        pltpu.sync_copy(w_hbm.at[wid], ws)

        def gather_start(j, b):
            pltpu.async_copy(table.at[srcs.at[j]], rows.at[b], sems_g[b])

        def gather_wait(j, b):
            pltpu.make_async_copy(table.at[srcs.at[j]], rows.at[b],
                                  sems_g[b]).wait()

        def scatter_start(j, b):
            pltpu.async_copy(rows.at[b], acc.at[dsts.at[j]], sems_s[b],
                             add=True)

        def scatter_wait(j, b):
            pltpu.make_async_copy(rows.at[b], acc.at[dsts.at[j]],
                                  sems_s[b]).wait()

        def compute(j, b):
            for g in range(K // 16):
                wv = ws[j, pl.ds(g * 16, 16)]
                for e in range(16):
                    wb = _broadcast_lane(wv, e)
                    ge = g * 16 + e
                    for f in range(nfv):
                        sl = pl.ds(f * 16, 16)
                        rows[b, ge, sl] = rows[b, ge, sl] * wb

        for p in range(npass):
            pltpu.sync_copy(h_hbm.at[p, pl.ds(r0, ROWS_T)],
                            table.at[pl.ds(r0, ROWS_T)])
            pltpu.sync_copy(zeros_hbm.at[pl.ds(r0, ROWS_T)],
                            acc.at[pl.ds(r0, ROWS_T)])

            @pl.when(s == NS - 1)
            def _():
                pltpu.sync_copy(h_hbm.at[p, rext], table.at[rext])
                pltpu.sync_copy(zeros_hbm.at[rext], acc.at[rext])

            plsc.subcore_barrier()

            gather_start(0, 0)

            def pair(jj, _):
                for b in range(2):
                    j = 2 * jj + b
                    gather_wait(j, b)

                    @pl.when(j >= 1)
                    def _():
                        scatter_wait(j - 1, 1 - b)

                    @pl.when(j + 1 < NCHUNK)
                    def _():
                        gather_start(j + 1, 1 - b)

                    compute(j, b)
                    scatter_start(j, b)
                return ()

            lax.fori_loop(0, NCHUNK // 2, pair, ())
            scatter_wait(NCHUNK - 1, 1)

            plsc.subcore_barrier()

            pltpu.sync_copy(acc.at[pl.ds(r0, ROWS_T)],
                            out_hbm.at[p, c, pl.ds(r0, ROWS_T)])

            @pl.when(s == NS - 1)
            def _():
                pltpu.sync_copy(acc.at[rext], out_hbm.at[p, c, rext])

    mesh = plsc.VectorSubcoreMesh(core_axis_name="c", subcore_axis_name="s")
    return pl.kernel(
        body,
        out_type=jax.ShapeDtypeStruct((npass, NC, N, feat), jnp.float32),
        mesh=mesh,
        compiler_params=pltpu.CompilerParams(use_tc_tiling_on_sc=False),
        scratch_types=[
            pltpu.VMEM_SHARED((N, feat), jnp.float32),
            pltpu.VMEM_SHARED((N, feat), jnp.float32),
            pltpu.VMEM((NCHUNK, K), jnp.int32),
            pltpu.VMEM((NCHUNK, K), jnp.int32),
            pltpu.VMEM((NCHUNK, K), jnp.float32),
            pltpu.VMEM((2, K, feat), jnp.float32),
            pltpu.SemaphoreType.DMA,
            pltpu.SemaphoreType.DMA,
            pltpu.SemaphoreType.DMA,
            pltpu.SemaphoreType.DMA,
        ],
    )


_sc_agg_64x2 = _make_sc_aggregate(LATENT // 2, 2)
_sc_agg_16 = _make_sc_aggregate(CLASSES, 1)

HALF = LATENT // 2


def _mm_body(x_ref, w_ref, o_ref):
    o_ref[...] = jnp.dot(x_ref[...], w_ref[...],
                         preferred_element_type=jnp.float32)


def _matmul_split(x, w):
    n, k = x.shape
    blk = 400

    def mm_body(x_ref, w_ref, o_ref):
        o_ref[0, ...] = jnp.dot(x_ref[...], w_ref[0],
                                preferred_element_type=jnp.float32)

    return pl.pallas_call(
        mm_body,
        grid=(n // blk, 2),
        in_specs=[pl.BlockSpec((blk, k), lambda i, j: (i, 0)),
                  pl.BlockSpec((1, k, HALF), lambda i, j: (j, 0, 0))],
        out_specs=pl.BlockSpec((1, blk, HALF), lambda i, j: (j, i, 0)),
        out_shape=jax.ShapeDtypeStruct((2, n, HALF), jnp.float32),
    )(x, w.reshape(k, 2, HALF).transpose(1, 0, 2))


def _combine1_body(pa0_ref, pa1_ref, pb0_ref, pb1_ref, b_ref,
                   w2a_ref, w2b_ref, out1_ref, ixz_ref, h2_ref):
    oa = pa0_ref[0, 0] + pa1_ref[0, 0] + b_ref[0, 0]
    ob = pb0_ref[0, 0] + pb1_ref[0, 0] + b_ref[0, 1]
    out1_ref[...] = jnp.concatenate([oa, ob], axis=1)
    std = jax.nn.softplus(ob) + 1e-10
    ixz_ref[...] = -jnp.log(std) + (std * std + oa * oa) / 2.0 - 0.5
    h2_ref[...] = (jnp.dot(oa, w2a_ref[0], preferred_element_type=jnp.float32)
                   + jnp.dot(ob, w2b_ref[0], preferred_element_type=jnp.float32))


def _combine1(p, b1, W2):
    blk = 400
    return pl.pallas_call(
        _combine1_body,
        grid=(N // blk,),
        in_specs=[pl.BlockSpec((1, 1, blk, HALF), lambda i: (0, 0, i, 0)),
                  pl.BlockSpec((1, 1, blk, HALF), lambda i: (0, 1, i, 0)),
                  pl.BlockSpec((1, 1, blk, HALF), lambda i: (1, 0, i, 0)),
                  pl.BlockSpec((1, 1, blk, HALF), lambda i: (1, 1, i, 0)),
                  pl.BlockSpec((1, 2, HALF), lambda i: (0, 0, 0)),
                  pl.BlockSpec((1, HALF, CLASSES), lambda i: (0, 0, 0)),
                  pl.BlockSpec((1, HALF, CLASSES), lambda i: (1, 0, 0))],
        out_specs=[pl.BlockSpec((blk, LATENT), lambda i: (i, 0)),
                   pl.BlockSpec((blk, HALF), lambda i: (i, 0)),
                   pl.BlockSpec((blk, CLASSES), lambda i: (i, 0))],
        out_shape=[jax.ShapeDtypeStruct((N, LATENT), jnp.float32),
                   jax.ShapeDtypeStruct((N, HALF), jnp.float32),
                   jax.ShapeDtypeStruct((N, CLASSES), jnp.float32)],
    )(p, p, p, p,
      b1.reshape(1, 2, HALF),
      W2.reshape(2, HALF, CLASSES), W2.reshape(2, HALF, CLASSES))


def _combine2_body(p0_ref, p1_ref, b_ref, out2_ref, ixz_ref):
    o = p0_ref[0, 0] + p1_ref[0, 0] + b_ref[...]
    out2_ref[...] = o
    mean = o[:, :CLASSES // 2]
    std = jax.nn.softplus(o[:, CLASSES // 2:]) + 1e-10
    ixz_ref[...] = -jnp.log(std) + (std * std + mean * mean) / 2.0 - 0.5


def _combine2(p, b2):
    blk = 1000
    return pl.pallas_call(
        _combine2_body,
        grid=(N // blk,),
        in_specs=[pl.BlockSpec((1, 1, blk, CLASSES), lambda i: (0, 0, i, 0)),
                  pl.BlockSpec((1, 1, blk, CLASSES), lambda i: (0, 1, i, 0)),
                  pl.BlockSpec((1, CLASSES), lambda i: (0, 0))],
        out_specs=[pl.BlockSpec((blk, CLASSES), lambda i: (i, 0)),
                   pl.BlockSpec((blk, CLASSES // 2), lambda i: (i, 0))],
        out_shape=[jax.ShapeDtypeStruct((N, CLASSES), jnp.float32),
                   jax.ShapeDtypeStruct((N, CLASSES // 2), jnp.float32)],
    )(p, p, b2.reshape(1, CLASSES))


def kernel(x, edge_index, edge_attr, W1, b1, W2, b2):
    pad = EPAD - E
    src = jnp.pad(edge_index[0].astype(jnp.int32), (0, pad)).reshape(NW, NCHUNK, K)
    dst = jnp.pad(edge_index[1].astype(jnp.int32), (0, pad)).reshape(NW, NCHUNK, K)
    w = jnp.pad(edge_attr.astype(jnp.float32), (0, pad)).reshape(NW, NCHUNK, K)
    zeros64 = jnp.zeros((N, HALF), jnp.float32)
    zeros16 = jnp.zeros((N, CLASSES), jnp.float32)

    h1s = _matmul_split(x, W1)
    p1 = _sc_agg_64x2(h1s, src, dst, w, zeros64)
    out1, ixz1, h2 = _combine1(p1, b1, W2)
    p2 = _sc_agg_16(h2[None], src, dst, w, zeros16)
    out2, ixz2 = _combine2(p2, b2)

    skl1 = jnp.zeros_like(ixz1)
    skl2 = jnp.zeros_like(ixz2)
    return (out2, out1, ixz1, skl1, ixz2, skl2)
